# Initial kernel scaffold; baseline (speedup 1.0000x reference)
import jax
import jax.numpy as jnp
from jax.experimental import pallas as pl


def kernel(input_ids, table, ln_gamma, ln_beta):
    raise NotImplementedError("write your pallas kernel here")


# speedup vs baseline: 1.1413x; 1.1413x over previous
import functools

import jax
import jax.numpy as jnp
from jax import lax
from jax.experimental import pallas as pl
from jax.experimental.pallas import tpu as pltpu
from jax.experimental.pallas import tpu_sc as plsc

EPS = 1e-12

NC = 2
NS = 16
NW = NC * NS


def _ln_table_body(t_ref, g_ref, b_ref, o_ref):
    t = t_ref[...]
    mean = jnp.mean(t, axis=1, keepdims=True)
    var = jnp.mean(jnp.square(t - mean), axis=1, keepdims=True)
    o_ref[...] = (t - mean) / jnp.sqrt(var + EPS) * g_ref[...] + b_ref[...]


def _normalize_table(table_p, gamma, beta):
    vp, h = table_p.shape
    return pl.pallas_call(
        _ln_table_body,
        out_shape=jax.ShapeDtypeStruct((vp, h), jnp.float32),
    )(table_p, gamma.reshape(1, h), beta.reshape(1, h))


def _make_gather(nt, d, chunk):
    bpw = nt // NW
    nch = bpw // chunk
    mesh = plsc.VectorSubcoreMesh(core_axis_name="c", subcore_axis_name="s")

    @functools.partial(
        pl.kernel,
        mesh=mesh,
        out_type=jax.ShapeDtypeStruct((nt, d), jnp.float32),
        scratch_types=[
            pltpu.VMEM((nch, chunk), jnp.int32),
            pltpu.VMEM((chunk, d), jnp.float32),
            pltpu.SemaphoreType.DMA,
        ],
    )
    def gather_k(idx_hbm, table_hbm, out_hbm, idx_v, rows_v, sem):
        wid = lax.axis_index("s") * NC + lax.axis_index("c")
        base = wid * bpw
        pltpu.sync_copy(idx_hbm.at[pl.ds(wid * nch, nch)], idx_v)
        for c in range(nch):
            pltpu.async_copy(table_hbm.at[idx_v.at[c]], rows_v, sem).wait()
            pltpu.sync_copy(rows_v, out_hbm.at[pl.ds(base + c * chunk, chunk)])

    return gather_k


def kernel(input_ids, table, ln_gamma, ln_beta):
    b, s = input_ids.shape
    v, h = table.shape
    nt = b * s

    vp = (v + 7) // 8 * 8
    table_p = jnp.pad(table, ((0, vp - v), (0, 0)))
    normed = _normalize_table(table_p, ln_gamma, ln_beta)

    chunk = 128
    ids2 = input_ids.reshape(nt // chunk, chunk).astype(jnp.int32)
    out = _make_gather(nt, h, chunk)(ids2, normed)
    return out.reshape(b, s, h)
